# baseline (speedup 1.0000x reference)
import jax
import jax.numpy as jnp
from jax import lax
from jax.experimental import pallas as pl
from jax.experimental.pallas import tpu as pltpu
from jax.experimental.pallas import tpu_sc as plsc

B, L, H = 32, 128, 512
G4 = 4 * H
N_TOK = B * L
NC, NS = 2, 16
NW = NC * NS
ROWS_PER_W = N_TOK // NW


def _sc_gather_body(table_hbm, idx_hbm, out_hbm, idx_v, rows_v, sem):
    wid = lax.axis_index("s") * NC + lax.axis_index("c")
    base = wid * ROWS_PER_W
    pltpu.sync_copy(idx_hbm.at[pl.ds(base, ROWS_PER_W)], idx_v)
    pltpu.async_copy(table_hbm.at[idx_v], rows_v, sem).wait()
    pltpu.sync_copy(rows_v, out_hbm.at[pl.ds(base, ROWS_PER_W)])


def _embed_gather(emb, idx):
    k = pl.kernel(
        _sc_gather_body,
        out_type=jax.ShapeDtypeStruct((N_TOK, H), jnp.float32),
        mesh=plsc.VectorSubcoreMesh(core_axis_name="c", subcore_axis_name="s"),
        scratch_types=[
            pltpu.VMEM((ROWS_PER_W,), jnp.int32),
            pltpu.VMEM((ROWS_PER_W, H), jnp.float32),
            pltpu.SemaphoreType.DMA,
        ],
    )
    return k(emb, idx)


_NEG = -3.0e38
_T_U = 32
_M_BLK = _T_U * B


def _sigmoid(x):
    return 0.5 * jnp.tanh(0.5 * x) + 0.5


def _fused_body(e_ref, wih_ref, b_ref, whh_ref, wcls_ref, bcls_ref,
                out_ref, gx_ref, h_ref, c_ref, p_ref):
    g = pl.program_id(0)

    @pl.when(g == 0)
    def _():
        h_ref[...] = jnp.zeros_like(h_ref)
        c_ref[...] = jnp.zeros_like(c_ref)
        p_ref[...] = jnp.full_like(p_ref, _NEG)

    gx_ref[...] = (
        jnp.dot(e_ref[0], wih_ref[...], preferred_element_type=jnp.float32)
        + b_ref[...]
    )

    h = h_ref[...]
    c = c_ref[...]
    p = p_ref[...]
    whh = whh_ref[...]
    for u in range(_T_U):
        h_bf = h.astype(jnp.bfloat16)
        base = u * B
        t_i = jnp.tanh(
            gx_ref[base : base + B, 0:H]
            + jnp.dot(h_bf, whh[:, 0:H], preferred_element_type=jnp.float32)
        )
        t_f = jnp.tanh(
            gx_ref[base : base + B, H : 2 * H]
            + jnp.dot(h_bf, whh[:, H : 2 * H], preferred_element_type=jnp.float32)
        )
        g_g = jnp.tanh(
            gx_ref[base : base + B, 2 * H : 3 * H]
            + jnp.dot(h_bf, whh[:, 2 * H : 3 * H], preferred_element_type=jnp.float32)
        )
        t_o = jnp.tanh(
            gx_ref[base : base + B, 3 * H : 4 * H]
            + jnp.dot(h_bf, whh[:, 3 * H : 4 * H], preferred_element_type=jnp.float32)
        )
        c = 0.5 * ((t_f + 1.0) * c + (t_i + 1.0) * g_g)
        h = 0.5 * (t_o + 1.0) * jnp.tanh(c)
        p = jnp.maximum(p, h)
    c_ref[...] = c
    h_ref[...] = h
    p_ref[...] = p

    @pl.when(g == L // _T_U - 1)
    def _():
        out_ref[...] = (
            jnp.dot(p, wcls_ref[...], preferred_element_type=jnp.float32)
            + bcls_ref[...]
        )


def _fused(e3, w_ihT, bias, w_hhT, w_clsT, b_cls2):
    return pl.pallas_call(
        _fused_body,
        grid=(L // _T_U,),
        in_specs=[
            pl.BlockSpec((1, _M_BLK, H), lambda g: (g, 0, 0)),
            pl.BlockSpec((H, G4), lambda g: (0, 0)),
            pl.BlockSpec((1, G4), lambda g: (0, 0)),
            pl.BlockSpec((H, G4), lambda g: (0, 0)),
            pl.BlockSpec((H, 2), lambda g: (0, 0)),
            pl.BlockSpec((1, 2), lambda g: (0, 0)),
        ],
        out_specs=pl.BlockSpec((B, 2), lambda g: (0, 0)),
        out_shape=jax.ShapeDtypeStruct((B, 2), jnp.float32),
        scratch_shapes=[
            pltpu.VMEM((_M_BLK, G4), jnp.float32),
            pltpu.VMEM((B, H), jnp.float32),
            pltpu.VMEM((B, H), jnp.float32),
            pltpu.VMEM((B, H), jnp.float32),
        ],
    )(e3, w_ihT, bias, w_hhT, w_clsT, b_cls2)


def kernel(x, emb, W_ih, W_hh, b_ih, b_hh, W_cls, b_cls):
    idx = x.astype(jnp.int32).T.reshape(-1)
    e = _embed_gather(emb, idx)
    e3 = e.reshape(L // _T_U, _M_BLK, H)
    scale = jnp.concatenate(
        [jnp.full((2 * H,), 0.5), jnp.ones((H,)), jnp.full((H,), 0.5)]
    ).astype(jnp.float32)
    bias = ((b_ih + b_hh) * scale).reshape(1, G4)
    w_ihT = W_ih.T * scale[None, :]
    w_hhT = (W_hh.T * scale[None, :]).astype(jnp.bfloat16)
    return _fused(e3, w_ihT, bias, w_hhT, W_cls.T, b_cls.reshape(1, 2))


# speedup vs baseline: 1.0001x; 1.0001x over previous
import jax
import jax.numpy as jnp
from jax import lax
from jax.experimental import pallas as pl
from jax.experimental.pallas import tpu as pltpu
from jax.experimental.pallas import tpu_sc as plsc

B, L, H = 32, 128, 512
G4 = 4 * H
N_TOK = B * L
NC, NS = 2, 16
NW = NC * NS
ROWS_PER_W = N_TOK // NW


def _sc_gather_body(table_hbm, idx_hbm, out_hbm, idx_v, rows_v, sem):
    wid = lax.axis_index("s") * NC + lax.axis_index("c")
    base = wid * ROWS_PER_W
    pltpu.sync_copy(idx_hbm.at[pl.ds(base, ROWS_PER_W)], idx_v)
    pltpu.async_copy(table_hbm.at[idx_v], rows_v, sem).wait()
    pltpu.sync_copy(rows_v, out_hbm.at[pl.ds(base, ROWS_PER_W)])


def _embed_gather(emb, idx):
    k = pl.kernel(
        _sc_gather_body,
        out_type=jax.ShapeDtypeStruct((N_TOK, H), jnp.float32),
        mesh=plsc.VectorSubcoreMesh(core_axis_name="c", subcore_axis_name="s"),
        scratch_types=[
            pltpu.VMEM((ROWS_PER_W,), jnp.int32),
            pltpu.VMEM((ROWS_PER_W, H), jnp.float32),
            pltpu.SemaphoreType.DMA,
        ],
    )
    return k(emb, idx)


_NEG = -3.0e38
_T_U = 32
_M_BLK = _T_U * B


def _sigmoid(x):
    return 0.5 * jnp.tanh(0.5 * x) + 0.5


def _fused_body(e_ref, wih_ref, b_ref, whh_ref, wcls_ref, bcls_ref,
                out_ref, gx_ref, h_ref, c_ref, p_ref):
    g = pl.program_id(0)

    @pl.when(g == 0)
    def _():
        h_ref[...] = jnp.zeros_like(h_ref)
        c_ref[...] = jnp.zeros_like(c_ref)
        p_ref[...] = jnp.full_like(p_ref, _NEG)

    gx_ref[...] = (
        jnp.dot(e_ref[0], wih_ref[...], preferred_element_type=jnp.float32)
        + b_ref[...]
    )

    h = h_ref[...]
    c = c_ref[...]
    p = p_ref[...]
    whh = whh_ref[...]
    for u in range(_T_U):
        gates = gx_ref[u * B : (u + 1) * B, :] + jnp.dot(
            h.astype(jnp.bfloat16), whh, preferred_element_type=jnp.float32
        )
        t_i = jnp.tanh(gates[:, 0:H])
        t_f = jnp.tanh(gates[:, H : 2 * H])
        g_g = jnp.tanh(gates[:, 2 * H : 3 * H])
        t_o = jnp.tanh(gates[:, 3 * H : 4 * H])
        c = 0.5 * ((t_f + 1.0) * c + (t_i + 1.0) * g_g)
        h = 0.5 * (t_o + 1.0) * jnp.tanh(c)
        p = jnp.maximum(p, h)
    c_ref[...] = c
    h_ref[...] = h
    p_ref[...] = p

    @pl.when(g == L // _T_U - 1)
    def _():
        out_ref[...] = (
            jnp.dot(p, wcls_ref[...], preferred_element_type=jnp.float32)
            + bcls_ref[...]
        )


def _fused(e3, w_ihT, bias, w_hhT, w_clsT, b_cls2):
    return pl.pallas_call(
        _fused_body,
        grid=(L // _T_U,),
        in_specs=[
            pl.BlockSpec((1, _M_BLK, H), lambda g: (g, 0, 0)),
            pl.BlockSpec((H, G4), lambda g: (0, 0)),
            pl.BlockSpec((1, G4), lambda g: (0, 0)),
            pl.BlockSpec((H, G4), lambda g: (0, 0)),
            pl.BlockSpec((H, 2), lambda g: (0, 0)),
            pl.BlockSpec((1, 2), lambda g: (0, 0)),
        ],
        out_specs=pl.BlockSpec((B, 2), lambda g: (0, 0)),
        out_shape=jax.ShapeDtypeStruct((B, 2), jnp.float32),
        scratch_shapes=[
            pltpu.VMEM((_M_BLK, G4), jnp.float32),
            pltpu.VMEM((B, H), jnp.float32),
            pltpu.VMEM((B, H), jnp.float32),
            pltpu.VMEM((B, H), jnp.float32),
        ],
    )(e3, w_ihT, bias, w_hhT, w_clsT, b_cls2)


def kernel(x, emb, W_ih, W_hh, b_ih, b_hh, W_cls, b_cls):
    idx = x.astype(jnp.int32).T.reshape(-1)
    e = _embed_gather(emb, idx)
    e3 = e.reshape(L // _T_U, _M_BLK, H)
    scale = jnp.concatenate(
        [jnp.full((2 * H,), 0.5), jnp.ones((H,)), jnp.full((H,), 0.5)]
    ).astype(jnp.float32)
    bias = ((b_ih + b_hh) * scale).reshape(1, G4)
    w_ihT = W_ih.T * scale[None, :]
    w_hhT = (W_hh.T * scale[None, :]).astype(jnp.bfloat16)
    return _fused(e3, w_ihT, bias, w_hhT, W_cls.T, b_cls.reshape(1, 2))


# speedup vs baseline: 1.0738x; 1.0737x over previous
import jax
import jax.numpy as jnp
from jax import lax
from jax.experimental import pallas as pl
from jax.experimental.pallas import tpu as pltpu
from jax.experimental.pallas import tpu_sc as plsc

B, L, H = 32, 128, 512
G4 = 4 * H
N_TOK = B * L
NC, NS = 2, 16
NW = NC * NS
ROWS_PER_W = N_TOK // NW


def _sc_gather_body(table_hbm, idx_hbm, out_hbm, idx_v, rows_v, sem):
    wid = lax.axis_index("s") * NC + lax.axis_index("c")
    base = wid * ROWS_PER_W
    pltpu.sync_copy(idx_hbm.at[pl.ds(base, ROWS_PER_W)], idx_v)
    pltpu.async_copy(table_hbm.at[idx_v], rows_v, sem).wait()
    pltpu.sync_copy(rows_v, out_hbm.at[pl.ds(base, ROWS_PER_W)])


def _embed_gather(emb, idx):
    k = pl.kernel(
        _sc_gather_body,
        out_type=jax.ShapeDtypeStruct((N_TOK, H), jnp.float32),
        mesh=plsc.VectorSubcoreMesh(core_axis_name="c", subcore_axis_name="s"),
        scratch_types=[
            pltpu.VMEM((ROWS_PER_W,), jnp.int32),
            pltpu.VMEM((ROWS_PER_W, H), jnp.float32),
            pltpu.SemaphoreType.DMA,
        ],
    )
    return k(emb, idx)


_NEG = -3.0e38
_T_U = 32
_M_BLK = _T_U * B


def _sigmoid(x):
    return 0.5 * jnp.tanh(0.5 * x) + 0.5


def _fused_body(e_ref, wih_ref, b_ref, whh_ref, wcls_ref, bcls_ref,
                out_ref, gx_ref, h_ref, c_ref, p_ref):
    g = pl.program_id(0)

    @pl.when(g == 0)
    def _():
        h_ref[...] = jnp.zeros_like(h_ref)
        c_ref[...] = jnp.zeros_like(c_ref)
        p_ref[...] = jnp.full_like(p_ref, _NEG)

    gx_ref[...] = (
        jnp.dot(e_ref[0], wih_ref[...], preferred_element_type=jnp.float32)
        + b_ref[...]
    )

    h = h_ref[...]
    c = c_ref[...]
    p = p_ref[...]
    whh = whh_ref[...]
    for u in range(_T_U):
        gates = gx_ref[u * B : (u + 1) * B, :] + jnp.dot(
            h.astype(jnp.bfloat16), whh, preferred_element_type=jnp.float32
        )
        i_g = _sigmoid(gates[:, 0:H])
        f_g = _sigmoid(gates[:, H : 2 * H])
        g_g = jnp.tanh(gates[:, 2 * H : 3 * H])
        o_g = _sigmoid(gates[:, 3 * H : 4 * H])
        c = f_g * c + i_g * g_g
        h = o_g * jnp.tanh(c)
        p = jnp.maximum(p, h)
    c_ref[...] = c
    h_ref[...] = h
    p_ref[...] = p

    @pl.when(g == L // _T_U - 1)
    def _():
        out_ref[...] = (
            jnp.dot(p, wcls_ref[...], preferred_element_type=jnp.float32)
            + bcls_ref[...]
        )


def _fused(e3, w_ihT, bias, w_hhT, w_clsT, b_cls2):
    return pl.pallas_call(
        _fused_body,
        grid=(L // _T_U,),
        in_specs=[
            pl.BlockSpec((1, _M_BLK, H), lambda g: (g, 0, 0)),
            pl.BlockSpec((H, G4), lambda g: (0, 0)),
            pl.BlockSpec((1, G4), lambda g: (0, 0)),
            pl.BlockSpec((H, G4), lambda g: (0, 0)),
            pl.BlockSpec((H, 2), lambda g: (0, 0)),
            pl.BlockSpec((1, 2), lambda g: (0, 0)),
        ],
        out_specs=pl.BlockSpec((B, 2), lambda g: (0, 0)),
        out_shape=jax.ShapeDtypeStruct((B, 2), jnp.float32),
        scratch_shapes=[
            pltpu.VMEM((_M_BLK, G4), jnp.float32),
            pltpu.VMEM((B, H), jnp.float32),
            pltpu.VMEM((B, H), jnp.float32),
            pltpu.VMEM((B, H), jnp.float32),
        ],
    )(e3, w_ihT, bias, w_hhT, w_clsT, b_cls2)


def kernel(x, emb, W_ih, W_hh, b_ih, b_hh, W_cls, b_cls):
    idx = x.astype(jnp.int32).T.reshape(-1)
    e = _embed_gather(emb, idx)
    e3 = e.reshape(L // _T_U, _M_BLK, H)
    bias = (b_ih + b_hh).reshape(1, G4)
    w_ihT = W_ih.T
    w_hhT = W_hh.T.astype(jnp.bfloat16)
    return _fused(e3, w_ihT, bias, w_hhT, W_cls.T, b_cls.reshape(1, 2))
